# baseline (device time: 15870 ns/iter reference)
import jax
import jax.numpy as jnp
from jax import lax
from jax.experimental import pallas as pl
from jax.experimental.pallas import tpu as pltpu

N_DEV = 8
K = 4


def kernel(x, t_emb, W_scale, W_shift):
    b, s, c_sh = x.shape
    c_full = c_sh * N_DEV
    s_blk = s // K
    eps = 1e-5
    n_steps = 2 * K + 1

    def body(x_ref, t_ref, ws_ref, wsh_ref, out_ref,
             stats_ref, comm_ref, minv_ref, mod_ref, send_sems, recv_sems):
        i = pl.program_id(0)
        my_i = lax.axis_index("i")

        @pl.when(i == 0)
        def _barrier():
            barrier_sem = pltpu.get_barrier_semaphore()
            for d in range(1, N_DEV):
                peer = lax.rem(my_i + d, N_DEV)
                pl.semaphore_signal(barrier_sem, inc=1, device_id=(peer,),
                                    device_id_type=pl.DeviceIdType.MESH)
            pl.semaphore_wait(barrier_sem, N_DEV - 1)

        @pl.when(i < K)
        def _stats():
            xb = x_ref[...]
            stats_ref[0, :, pl.ds(i * s_blk, s_blk)] = jnp.sum(xb, axis=-1)
            stats_ref[1, :, pl.ds(i * s_blk, s_blk)] = jnp.sum(xb * xb, axis=-1)

        @pl.when(i == K)
        def _exchange():
            sends = []
            for d in range(1, N_DEV):
                target = lax.rem(my_i + d, N_DEV)
                rdma = pltpu.make_async_remote_copy(
                    src_ref=stats_ref,
                    dst_ref=comm_ref.at[d - 1],
                    send_sem=send_sems.at[d - 1],
                    recv_sem=recv_sems.at[d - 1],
                    device_id=(target,),
                    device_id_type=pl.DeviceIdType.MESH,
                )
                rdma.start()
                sends.append(rdma)

            mod_ref[0, :, :] = 1.0 + jnp.dot(
                t_ref[...], ws_ref[...], preferred_element_type=jnp.float32)
            mod_ref[1, :, :] = jnp.dot(
                t_ref[...], wsh_ref[...], preferred_element_type=jnp.float32)

            total = stats_ref[...]
            for d in range(1, N_DEV):
                sends[d - 1].wait_recv()
                total = total + comm_ref[d - 1]
            for d in range(1, N_DEV):
                sends[d - 1].wait_send()

            mean = total[0] / c_full
            var = total[1] / c_full - mean * mean
            minv_ref[0, :, :] = mean
            minv_ref[1, :, :] = lax.rsqrt(var + eps)

        @pl.when(i > K)
        def _apply():
            k = i - (K + 1)
            xb = x_ref[...]
            mean = minv_ref[0, :, pl.ds(k * s_blk, s_blk)]
            inv = minv_ref[1, :, pl.ds(k * s_blk, s_blk)]
            h = (xb - mean[:, :, None]) * inv[:, :, None]
            sc = mod_ref[0]
            sh = mod_ref[1]
            out_ref[...] = h * sc[:, None, :] + sh[:, None, :]

    def x_index(i):
        return (0, jnp.where(i < K, i, jnp.where(i > K, i - (K + 1), K - 1)), 0)

    def out_index(i):
        return (0, jnp.where(i <= K + 1, 0, i - (K + 1)), 0)

    return pl.pallas_call(
        body,
        grid=(n_steps,),
        out_shape=jax.ShapeDtypeStruct((b, s, c_sh), jnp.float32),
        in_specs=[
            pl.BlockSpec((b, s_blk, c_sh), x_index),
            pl.BlockSpec((4, 128), lambda i: (0, 0)),
            pl.BlockSpec((128, c_sh), lambda i: (0, 0)),
            pl.BlockSpec((128, c_sh), lambda i: (0, 0)),
        ],
        out_specs=pl.BlockSpec((b, s_blk, c_sh), out_index),
        scratch_shapes=[
            pltpu.VMEM((2, b, s), jnp.float32),
            pltpu.VMEM((N_DEV - 1, 2, b, s), jnp.float32),
            pltpu.VMEM((2, b, s), jnp.float32),
            pltpu.VMEM((2, b, c_sh), jnp.float32),
            pltpu.SemaphoreType.DMA((N_DEV - 1,)),
            pltpu.SemaphoreType.DMA((N_DEV - 1,)),
        ],
        compiler_params=pltpu.CompilerParams(
            collective_id=0,
            dimension_semantics=("arbitrary",),
        ),
    )(x, t_emb, W_scale, W_shift)


# device time: 15735 ns/iter; 1.0086x vs baseline; 1.0086x over previous
import jax
import jax.numpy as jnp
from jax import lax
from jax.experimental import pallas as pl
from jax.experimental.pallas import tpu as pltpu

N_DEV = 8
K = 4


def kernel(x, t_emb, W_scale, W_shift):
    b, s, c_sh = x.shape
    c_full = c_sh * N_DEV
    s_blk = s // K
    eps = 1e-5
    n_steps = 2 * K + 1

    def body(x_ref, t_ref, ws_ref, wsh_ref, out_ref,
             stats_ref, comm_ref, minv_ref, mod_ref, xsave_ref,
             send_sems, recv_sems):
        i = pl.program_id(0)
        my_i = lax.axis_index("i")

        @pl.when(i == 0)
        def _barrier():
            barrier_sem = pltpu.get_barrier_semaphore()
            for d in range(1, N_DEV):
                peer = lax.rem(my_i + d, N_DEV)
                pl.semaphore_signal(barrier_sem, inc=1, device_id=(peer,),
                                    device_id_type=pl.DeviceIdType.MESH)
            pl.semaphore_wait(barrier_sem, N_DEV - 1)

        @pl.when(i < K)
        def _stats():
            xb = x_ref[...]
            stats_ref[0, :, pl.ds(i * s_blk, s_blk)] = jnp.sum(xb, axis=-1)
            stats_ref[1, :, pl.ds(i * s_blk, s_blk)] = jnp.sum(xb * xb, axis=-1)
            xsave_ref[i] = xb

        @pl.when(i == K)
        def _exchange():
            sends = []
            for d in range(1, N_DEV):
                target = lax.rem(my_i + d, N_DEV)
                rdma = pltpu.make_async_remote_copy(
                    src_ref=stats_ref,
                    dst_ref=comm_ref.at[d - 1],
                    send_sem=send_sems.at[d - 1],
                    recv_sem=recv_sems.at[d - 1],
                    device_id=(target,),
                    device_id_type=pl.DeviceIdType.MESH,
                )
                rdma.start()
                sends.append(rdma)

            mod_ref[0, :, :] = 1.0 + jnp.dot(
                t_ref[...], ws_ref[...], preferred_element_type=jnp.float32)
            mod_ref[1, :, :] = jnp.dot(
                t_ref[...], wsh_ref[...], preferred_element_type=jnp.float32)

            total = stats_ref[...]
            for d in range(1, N_DEV):
                sends[d - 1].wait_recv()
                total = total + comm_ref[d - 1]
            for d in range(1, N_DEV):
                sends[d - 1].wait_send()

            mean = total[0] / c_full
            var = total[1] / c_full - mean * mean
            minv_ref[0, :, :] = mean
            minv_ref[1, :, :] = lax.rsqrt(var + eps)

        @pl.when(i > K)
        def _apply():
            k = i - (K + 1)
            xb = xsave_ref[k]
            mean = minv_ref[0, :, pl.ds(k * s_blk, s_blk)]
            inv = minv_ref[1, :, pl.ds(k * s_blk, s_blk)]
            h = (xb - mean[:, :, None]) * inv[:, :, None]
            sc = mod_ref[0]
            sh = mod_ref[1]
            out_ref[...] = h * sc[:, None, :] + sh[:, None, :]

    def x_index(i):
        return (0, jnp.where(i < K, i, K - 1), 0)

    def out_index(i):
        return (0, jnp.where(i <= K + 1, 0, i - (K + 1)), 0)

    return pl.pallas_call(
        body,
        grid=(n_steps,),
        out_shape=jax.ShapeDtypeStruct((b, s, c_sh), jnp.float32),
        in_specs=[
            pl.BlockSpec((b, s_blk, c_sh), x_index),
            pl.BlockSpec((4, 128), lambda i: (0, 0)),
            pl.BlockSpec((128, c_sh), lambda i: (0, 0)),
            pl.BlockSpec((128, c_sh), lambda i: (0, 0)),
        ],
        out_specs=pl.BlockSpec((b, s_blk, c_sh), out_index),
        scratch_shapes=[
            pltpu.VMEM((2, b, s), jnp.float32),
            pltpu.VMEM((N_DEV - 1, 2, b, s), jnp.float32),
            pltpu.VMEM((2, b, s), jnp.float32),
            pltpu.VMEM((2, b, c_sh), jnp.float32),
            pltpu.VMEM((K, b, s_blk, c_sh), jnp.float32),
            pltpu.SemaphoreType.DMA((N_DEV - 1,)),
            pltpu.SemaphoreType.DMA((N_DEV - 1,)),
        ],
        compiler_params=pltpu.CompilerParams(
            collective_id=0,
            dimension_semantics=("arbitrary",),
        ),
    )(x, t_emb, W_scale, W_shift)


# device time: 15372 ns/iter; 1.0324x vs baseline; 1.0236x over previous
import jax
import jax.numpy as jnp
from jax import lax
from jax.experimental import pallas as pl
from jax.experimental.pallas import tpu as pltpu

N_DEV = 8
K = 4


def kernel(x, t_emb, W_scale, W_shift):
    b, s, c_sh = x.shape
    c_full = c_sh * N_DEV
    s_blk = s // K
    eps = 1e-5

    def body(x_hbm, t_ref, ws_ref, wsh_ref, out_hbm,
             xv_ref, ov_ref, stats_ref, comm_ref, mod_ref,
             in_sems, out_sems, send_sems, recv_sems):
        my_i = lax.axis_index("i")

        def seq(k):
            return (slice(None), pl.ds(k * s_blk, s_blk), slice(None))

        cin = []
        for k in range(K):
            c = pltpu.make_async_copy(
                x_hbm.at[seq(k)], xv_ref.at[seq(k)], in_sems.at[k])
            c.start()
            cin.append(c)

        barrier_sem = pltpu.get_barrier_semaphore()
        for d in range(1, N_DEV):
            peer = lax.rem(my_i + d, N_DEV)
            pl.semaphore_signal(barrier_sem, inc=1, device_id=(peer,),
                                device_id_type=pl.DeviceIdType.MESH)
        pl.semaphore_wait(barrier_sem, N_DEV - 1)

        mod_ref[0, :, :] = 1.0 + jnp.dot(
            t_ref[...], ws_ref[...], preferred_element_type=jnp.float32)
        mod_ref[1, :, :] = jnp.dot(
            t_ref[...], wsh_ref[...], preferred_element_type=jnp.float32)

        for k in range(K):
            cin[k].wait()
            xb = xv_ref[seq(k)]
            stats_ref[0, :, pl.ds(k * s_blk, s_blk)] = jnp.sum(xb, axis=-1)
            stats_ref[1, :, pl.ds(k * s_blk, s_blk)] = jnp.sum(xb * xb, axis=-1)

        sends = []
        for d in range(1, N_DEV):
            target = lax.rem(my_i + d, N_DEV)
            rdma = pltpu.make_async_remote_copy(
                src_ref=stats_ref,
                dst_ref=comm_ref.at[d - 1],
                send_sem=send_sems.at[d - 1],
                recv_sem=recv_sems.at[d - 1],
                device_id=(target,),
                device_id_type=pl.DeviceIdType.MESH,
            )
            rdma.start()
            sends.append(rdma)

        total = stats_ref[...]
        for d in range(1, N_DEV):
            sends[d - 1].wait_recv()
            total = total + comm_ref[d - 1]
        for d in range(1, N_DEV):
            sends[d - 1].wait_send()

        mean = total[0] / c_full
        var = total[1] / c_full - mean * mean
        inv = lax.rsqrt(var + eps)
        sc = mod_ref[0]
        sh = mod_ref[1]

        cout = []
        for k in range(K):
            xb = xv_ref[seq(k)]
            m = mean[:, k * s_blk:(k + 1) * s_blk][:, :, None]
            iv = inv[:, k * s_blk:(k + 1) * s_blk][:, :, None]
            ov_ref[seq(k)] = ((xb - m) * iv) * sc[:, None, :] + sh[:, None, :]
            c = pltpu.make_async_copy(
                ov_ref.at[seq(k)], out_hbm.at[seq(k)], out_sems.at[k])
            c.start()
            cout.append(c)
        for k in range(K):
            cout[k].wait()

    return pl.pallas_call(
        body,
        out_shape=jax.ShapeDtypeStruct((b, s, c_sh), jnp.float32),
        in_specs=[
            pl.BlockSpec(memory_space=pl.ANY),
            pl.BlockSpec(memory_space=pltpu.VMEM),
            pl.BlockSpec(memory_space=pltpu.VMEM),
            pl.BlockSpec(memory_space=pltpu.VMEM),
        ],
        out_specs=pl.BlockSpec(memory_space=pl.ANY),
        scratch_shapes=[
            pltpu.VMEM((b, s, c_sh), jnp.float32),
            pltpu.VMEM((b, s, c_sh), jnp.float32),
            pltpu.VMEM((2, b, s), jnp.float32),
            pltpu.VMEM((N_DEV - 1, 2, b, s), jnp.float32),
            pltpu.VMEM((2, b, c_sh), jnp.float32),
            pltpu.SemaphoreType.DMA((K,)),
            pltpu.SemaphoreType.DMA((K,)),
            pltpu.SemaphoreType.DMA((N_DEV - 1,)),
            pltpu.SemaphoreType.DMA((N_DEV - 1,)),
        ],
        compiler_params=pltpu.CompilerParams(collective_id=0),
    )(x, t_emb, W_scale, W_shift)


# device time: 14784 ns/iter; 1.0735x vs baseline; 1.0398x over previous
import jax
import jax.numpy as jnp
from jax import lax
from jax.experimental import pallas as pl
from jax.experimental.pallas import tpu as pltpu

N_DEV = 8
K = 4


def kernel(x, t_emb, W_scale, W_shift):
    b, s, c_sh = x.shape
    c_full = c_sh * N_DEV
    s_blk = s // K
    eps = 1e-5

    def body(x_ref, t_ref, ws_ref, wsh_ref, out_hbm,
             stats_ref, comm_ref, ov_ref, send_sems, recv_sems, out_sems):
        my_i = lax.axis_index("i")

        xv = x_ref[...]
        stats_ref[0, :, :] = jnp.sum(xv, axis=-1)
        stats_ref[1, :, :] = jnp.sum(xv * xv, axis=-1)

        barrier_sem = pltpu.get_barrier_semaphore()
        for d in range(1, N_DEV):
            peer = lax.rem(my_i + d, N_DEV)
            pl.semaphore_signal(barrier_sem, inc=1, device_id=(peer,),
                                device_id_type=pl.DeviceIdType.MESH)
        pl.semaphore_wait(barrier_sem, N_DEV - 1)

        sends = []
        for d in range(1, N_DEV):
            target = lax.rem(my_i + d, N_DEV)
            rdma = pltpu.make_async_remote_copy(
                src_ref=stats_ref,
                dst_ref=comm_ref.at[d - 1],
                send_sem=send_sems.at[d - 1],
                recv_sem=recv_sems.at[d - 1],
                device_id=(target,),
                device_id_type=pl.DeviceIdType.MESH,
            )
            rdma.start()
            sends.append(rdma)

        scale = jnp.dot(t_ref[...], ws_ref[...],
                        preferred_element_type=jnp.float32)
        shift = jnp.dot(t_ref[...], wsh_ref[...],
                        preferred_element_type=jnp.float32)

        total = stats_ref[...]
        for d in range(1, N_DEV):
            sends[d - 1].wait_recv()
            total = total + comm_ref[d - 1]
        for d in range(1, N_DEV):
            sends[d - 1].wait_send()

        mean = total[0] / c_full
        var = total[1] / c_full - mean * mean
        inv = lax.rsqrt(var + eps)
        sc = 1.0 + scale[:, None, :]
        sh = shift[:, None, :]

        cout = []
        for k in range(K):
            rows = slice(k * s_blk, (k + 1) * s_blk)
            xb = xv[:, rows, :]
            m = mean[:, rows][:, :, None]
            iv = inv[:, rows][:, :, None]
            ov_ref[:, rows, :] = ((xb - m) * iv) * sc + sh
            c = pltpu.make_async_copy(
                ov_ref.at[:, rows, :], out_hbm.at[:, rows, :], out_sems.at[k])
            c.start()
            cout.append(c)
        for k in range(K):
            cout[k].wait()

    return pl.pallas_call(
        body,
        out_shape=jax.ShapeDtypeStruct((b, s, c_sh), jnp.float32),
        in_specs=[pl.BlockSpec(memory_space=pltpu.VMEM)] * 4,
        out_specs=pl.BlockSpec(memory_space=pl.ANY),
        scratch_shapes=[
            pltpu.VMEM((2, b, s), jnp.float32),
            pltpu.VMEM((N_DEV - 1, 2, b, s), jnp.float32),
            pltpu.VMEM((b, s, c_sh), jnp.float32),
            pltpu.SemaphoreType.DMA((N_DEV - 1,)),
            pltpu.SemaphoreType.DMA((N_DEV - 1,)),
            pltpu.SemaphoreType.DMA((K,)),
        ],
        compiler_params=pltpu.CompilerParams(collective_id=0),
    )(x, t_emb, W_scale, W_shift)
